# baseline (speedup 1.0000x reference)
import functools

import jax
import jax.numpy as jnp
from jax import lax
from jax.experimental import pallas as pl
from jax.experimental.pallas import tpu as pltpu
from jax.experimental.pallas import tpu_sc as plsc

_N = 10000
_E = 160000
_D_IN = 256
_H = 16
_CLS = 41

_NC = 2
_NS = 16
_NW = _NC * _NS
_CH = 128
_EPAD = 163840
_NCHT = _EPAD // _CH
_NCH0 = 56
_NCH1 = 24
_C1OFF = _NS * _NCH0
_NPADR = 10112
_RPT = _NPADR // _NS
_NBUF = 8
_PREF = 3


def _make_seg(with_deg):
    mesh = plsc.VectorSubcoreMesh(core_axis_name="c", subcore_axis_name="s",
                                  num_cores=_NC, num_subcores=_NS)
    out_type = [jax.ShapeDtypeStruct((_NC * _NPADR, _H), jnp.float32)]
    scratch = [
        pltpu.VMEM((_NCH0, _CH), jnp.int32),
        pltpu.VMEM((_NCH0, _CH), jnp.int32),
        pltpu.VMEM((_NBUF, _CH, _H), jnp.float32),
        pltpu.VMEM_SHARED((_NPADR, _H), jnp.float32),
        pltpu.SemaphoreType.DMA,
        [pltpu.SemaphoreType.DMA] * _NBUF,
        [pltpu.SemaphoreType.DMA] * _NBUF,
    ]
    if with_deg:
        out_type.append(jax.ShapeDtypeStruct((_NC * _NPADR,), jnp.float32))
        scratch += [
            pltpu.VMEM((_CH,), jnp.float32),
            pltpu.VMEM_SHARED((_NPADR,), jnp.float32),
            pltpu.SemaphoreType.DMA,
        ]

    def body(*refs):
        if with_deg:
            (table, srcp, dstp, z2, z1, out, deg_out,
             src_v, dst_v, rows_v, acc_sh, isem, gsem, ssem,
             ones_v, deg_sh, dsem) = refs
        else:
            (table, srcp, dstp, z2, out,
             src_v, dst_v, rows_v, acc_sh, isem, gsem, ssem) = refs
        cid = lax.axis_index("c")
        sid = lax.axis_index("s")
        r0 = sid * _RPT
        nch = jnp.where(cid == 0, _NCH0, _NCH1)

        @pl.when(cid == 0)
        def _():
            pltpu.async_copy(srcp.at[pl.ds(sid * _NCH0, _NCH0)],
                             src_v, isem)
            pltpu.async_copy(dstp.at[pl.ds(sid * _NCH0, _NCH0)],
                             dst_v, isem)
            pltpu.make_async_copy(srcp.at[pl.ds(0, _NCH0)], src_v,
                                  isem).wait()
            pltpu.make_async_copy(dstp.at[pl.ds(0, _NCH0)], dst_v,
                                  isem).wait()

        @pl.when(cid == 1)
        def _():
            pltpu.async_copy(srcp.at[pl.ds(_C1OFF + sid * _NCH1, _NCH1)],
                             src_v.at[pl.ds(0, _NCH1)], isem)
            pltpu.async_copy(dstp.at[pl.ds(_C1OFF + sid * _NCH1, _NCH1)],
                             dst_v.at[pl.ds(0, _NCH1)], isem)
            pltpu.make_async_copy(srcp.at[pl.ds(0, _NCH1)],
                                  src_v.at[pl.ds(0, _NCH1)], isem).wait()
            pltpu.make_async_copy(dstp.at[pl.ds(0, _NCH1)],
                                  dst_v.at[pl.ds(0, _NCH1)], isem).wait()
        if with_deg:
            @pl.when(sid == 0)
            def _():
                pltpu.async_copy(z1, deg_sh, isem).wait()
            for i in range(_CH // 16):
                ones_v[pl.ds(i * 16, 16)] = jnp.ones((16,), jnp.float32)
        pltpu.sync_copy(z2.at[pl.ds(r0, _RPT)], acc_sh.at[pl.ds(r0, _RPT)])
        plsc.subcore_barrier()

        def gather(jj, bb):
            pltpu.async_copy(table.at[src_v.at[jj]], rows_v.at[bb], gsem[bb])

        for b in range(_PREF):
            gather(b, b)

        def step(g, carry):
            for b in range(_NBUF):
                j = g * _NBUF + b

                @pl.when(j <= nch - 1)
                def _():
                    pltpu.make_async_copy(table.at[src_v.at[j]],
                                          rows_v.at[b], gsem[b]).wait()
                    pltpu.async_copy(rows_v.at[b], acc_sh.at[dst_v.at[j]],
                                     ssem[b], add=True)
                    if with_deg:
                        pltpu.async_copy(ones_v, deg_sh.at[dst_v.at[j]],
                                         dsem, add=True)

                        @pl.when(j >= 1)
                        def _():
                            pltpu.make_async_copy(
                                ones_v, deg_sh.at[dst_v.at[0]], dsem).wait()
                    jj = j + _PREF
                    bb = (b + _PREF) % _NBUF

                    @pl.when(jj <= nch - 1)
                    def _():
                        @pl.when(jj >= _NBUF)
                        def _():
                            pltpu.make_async_copy(
                                rows_v.at[bb], acc_sh.at[dst_v.at[0]],
                                ssem[bb]).wait()
                        gather(jj, bb)
            return carry

        lax.fori_loop(0, _NCH0 // _NBUF, step, 0)

        for b in range(_NBUF):
            pltpu.make_async_copy(rows_v.at[b], acc_sh.at[dst_v.at[0]],
                                  ssem[b]).wait()
        if with_deg:
            pltpu.make_async_copy(ones_v, deg_sh.at[dst_v.at[0]], dsem).wait()
        plsc.subcore_barrier()

        pltpu.sync_copy(acc_sh.at[pl.ds(r0, _RPT)],
                        out.at[pl.ds(cid * _NPADR + r0, _RPT)])
        if with_deg:
            @pl.when(sid == 0)
            def _():
                pltpu.sync_copy(deg_sh, deg_out.at[pl.ds(cid * _NPADR,
                                                         _NPADR)])

    return pl.kernel(
        body,
        out_type=tuple(out_type) if with_deg else out_type[0],
        mesh=mesh,
        scratch_types=scratch,
        compiler_params=pltpu.CompilerParams(use_tc_tiling_on_sc=False),
    )


@functools.lru_cache(maxsize=None)
def _get_seg(with_deg):
    return _make_seg(with_deg)


_PBLK = _RPT
_NBLK = _NPADR // _PBLK


def _proj_body(x_ref, w_ref, xl_ref, xr_ref):
    res = jnp.dot(x_ref[...], w_ref[...], preferred_element_type=jnp.float32)
    xl_ref[...] = res[:, :_H]
    xr_ref[...] = res[:, _H:]


def _proj(x, wcat):
    row = lambda i: (i, 0)
    return pl.pallas_call(
        _proj_body,
        grid=(_NBLK,),
        in_specs=[
            pl.BlockSpec((_PBLK, _D_IN), row),
            pl.BlockSpec((_D_IN, 2 * _H), lambda i: (0, 0)),
        ],
        out_specs=[
            pl.BlockSpec((_PBLK, _H), row),
            pl.BlockSpec((_PBLK, _H), row),
        ],
        out_shape=[
            jax.ShapeDtypeStruct((_NPADR, _H), jnp.float32),
            jax.ShapeDtypeStruct((_NPADR, _H), jnp.float32),
        ],
    )(x, wcat)


def _combine_body(p_ref0, p_ref1, d_ref0, d_ref1, xr_ref, b1_ref, h_ref):
    deg = jnp.maximum(d_ref0[...] + d_ref1[...], 1.0)
    mean = (p_ref0[...] + p_ref1[...]) / deg
    h_ref[...] = jnp.maximum(mean + b1_ref[...] + xr_ref[...], 0.0)


def _combine(p, degs2, xr, b1):
    row = lambda i: (i, 0)
    row1 = lambda i: (_NBLK + i, 0)
    return pl.pallas_call(
        _combine_body,
        grid=(_NBLK,),
        in_specs=[
            pl.BlockSpec((_PBLK, _H), row),
            pl.BlockSpec((_PBLK, _H), row1),
            pl.BlockSpec((_PBLK, 1), row),
            pl.BlockSpec((_PBLK, 1), row1),
            pl.BlockSpec((_PBLK, _H), row),
            pl.BlockSpec((1, _H), lambda i: (0, 0)),
        ],
        out_specs=pl.BlockSpec((_PBLK, _H), row),
        out_shape=jax.ShapeDtypeStruct((_NPADR, _H), jnp.float32),
    )(p, p, degs2, degs2, xr, b1)


def _final_body(q_ref0, q_ref1, d_ref0, d_ref1, h_ref, wl_ref, wr_ref,
                b2_ref, o_ref):
    deg = jnp.maximum(d_ref0[...] + d_ref1[...], 1.0)
    mean2 = (q_ref0[...] + q_ref1[...]) / deg
    logits = (jnp.dot(mean2, wl_ref[...], preferred_element_type=jnp.float32)
              + jnp.dot(h_ref[...], wr_ref[...],
                        preferred_element_type=jnp.float32)
              + b2_ref[...])
    m = jnp.max(logits, axis=1, keepdims=True)
    ex = jnp.exp(logits - m)
    s = jnp.sum(ex, axis=1, keepdims=True)
    o_ref[...] = logits - m - jnp.log(s)


def _final(q, degs2, h, wl, wr, b2):
    row = lambda i: (i, 0)
    row1 = lambda i: (_NBLK + i, 0)
    full = lambda i: (0, 0)
    return pl.pallas_call(
        _final_body,
        grid=(_NBLK,),
        in_specs=[
            pl.BlockSpec((_PBLK, _H), row),
            pl.BlockSpec((_PBLK, _H), row1),
            pl.BlockSpec((_PBLK, 1), row),
            pl.BlockSpec((_PBLK, 1), row1),
            pl.BlockSpec((_PBLK, _H), row),
            pl.BlockSpec((_H, _CLS), full),
            pl.BlockSpec((_H, _CLS), full),
            pl.BlockSpec((1, _CLS), full),
        ],
        out_specs=pl.BlockSpec((_PBLK, _CLS), row),
        out_shape=jax.ShapeDtypeStruct((_N, _CLS), jnp.float32),
    )(q, q, degs2, degs2, h, wl, wr, b2)


def kernel(x, edge_index, W1_l, b1, W1_r, W2_l, b2, W2_r):
    src = edge_index[0]
    dst = edge_index[1]
    pad = _EPAD - _E
    srcp = jnp.concatenate([src, jnp.zeros((pad,), jnp.int32)]
                           ).reshape(_NCHT, _CH)
    dstp = jnp.concatenate([dst, jnp.full((pad,), _N, jnp.int32)]
                           ).reshape(_NCHT, _CH)
    z2 = jnp.zeros((_NPADR, _H), jnp.float32)
    z1 = jnp.zeros((_NPADR,), jnp.float32)

    wcat1 = jnp.concatenate([W1_l.T, W1_r.T], axis=1)
    xl, xr = _proj(x, wcat1)

    p, degs = _get_seg(True)(xl, srcp, dstp, z2, z1)
    degs2 = degs.reshape(_NC * _NPADR, 1)
    h = _combine(p, degs2, xr, b1.reshape(1, _H))

    q = _get_seg(False)(h, srcp, dstp, z2)
    return _final(q, degs2, h, W2_l.T, W2_r.T, b2.reshape(1, _CLS))


# speedup vs baseline: 1.2047x; 1.2047x over previous
import functools

import jax
import jax.numpy as jnp
from jax import lax
from jax.experimental import pallas as pl
from jax.experimental.pallas import tpu as pltpu
from jax.experimental.pallas import tpu_sc as plsc

_N = 10000
_E = 160000
_D_IN = 256
_H = 16
_CLS = 41

_NC = 2
_NS = 16
_NW = _NC * _NS
_CH = 128
_EPAD = 163840
_NCHT = _EPAD // _CH
_NCH0 = 40
_NCH1 = 40
_C1OFF = _NS * _NCH0
_NPADR = 10112
_RPT = _NPADR // _NS
_NBUF = 8
_PREF = 3


def _make_seg(with_deg):
    mesh = plsc.VectorSubcoreMesh(core_axis_name="c", subcore_axis_name="s",
                                  num_cores=_NC, num_subcores=_NS)
    out_type = [jax.ShapeDtypeStruct((_NC * _NPADR, _H), jnp.float32)]
    scratch = [
        pltpu.VMEM((_NCH0, _CH), jnp.int32),
        pltpu.VMEM((_NCH0, _CH), jnp.int32),
        pltpu.VMEM((_NBUF, _CH, _H), jnp.float32),
        pltpu.VMEM_SHARED((_NPADR, _H), jnp.float32),
        pltpu.VMEM_SHARED((_NPADR, _H), jnp.float32),
        pltpu.SemaphoreType.DMA,
        [pltpu.SemaphoreType.DMA] * _NBUF,
        [pltpu.SemaphoreType.DMA] * _NBUF,
    ]
    if with_deg:
        out_type.append(jax.ShapeDtypeStruct((_NC * _NPADR,), jnp.float32))
        scratch += [
            pltpu.VMEM((_CH,), jnp.float32),
            pltpu.VMEM_SHARED((_NPADR,), jnp.float32),
            pltpu.SemaphoreType.DMA,
        ]

    def body(*refs):
        if with_deg:
            (table, srcp, dstp, z2, z1, out, deg_out,
             src_v, dst_v, rows_v, acc_sh, table_sh, isem, gsem, ssem,
             ones_v, deg_sh, dsem) = refs
        else:
            (table, srcp, dstp, z2, out,
             src_v, dst_v, rows_v, acc_sh, table_sh, isem, gsem, ssem) = refs
        cid = lax.axis_index("c")
        sid = lax.axis_index("s")
        r0 = sid * _RPT
        nch = jnp.where(cid == 0, _NCH0, _NCH1)

        @pl.when(cid == 0)
        def _():
            pltpu.async_copy(srcp.at[pl.ds(sid * _NCH0, _NCH0)],
                             src_v, isem)
            pltpu.async_copy(dstp.at[pl.ds(sid * _NCH0, _NCH0)],
                             dst_v, isem)
            pltpu.make_async_copy(srcp.at[pl.ds(0, _NCH0)], src_v,
                                  isem).wait()
            pltpu.make_async_copy(dstp.at[pl.ds(0, _NCH0)], dst_v,
                                  isem).wait()

        @pl.when(cid == 1)
        def _():
            pltpu.async_copy(srcp.at[pl.ds(_C1OFF + sid * _NCH1, _NCH1)],
                             src_v.at[pl.ds(0, _NCH1)], isem)
            pltpu.async_copy(dstp.at[pl.ds(_C1OFF + sid * _NCH1, _NCH1)],
                             dst_v.at[pl.ds(0, _NCH1)], isem)
            pltpu.make_async_copy(srcp.at[pl.ds(0, _NCH1)],
                                  src_v.at[pl.ds(0, _NCH1)], isem).wait()
            pltpu.make_async_copy(dstp.at[pl.ds(0, _NCH1)],
                                  dst_v.at[pl.ds(0, _NCH1)], isem).wait()
        if with_deg:
            @pl.when(sid == 0)
            def _():
                pltpu.async_copy(z1, deg_sh, isem).wait()
            for i in range(_CH // 16):
                ones_v[pl.ds(i * 16, 16)] = jnp.ones((16,), jnp.float32)
        ld_tb = pltpu.async_copy(table.at[pl.ds(r0, _RPT)],
                                 table_sh.at[pl.ds(r0, _RPT)], isem)
        pltpu.sync_copy(z2.at[pl.ds(r0, _RPT)], acc_sh.at[pl.ds(r0, _RPT)])
        ld_tb.wait()
        plsc.subcore_barrier()

        def gather(jj, bb):
            pltpu.async_copy(table_sh.at[src_v.at[jj]], rows_v.at[bb],
                             gsem[bb])

        for b in range(_PREF):
            gather(b, b)

        def step(g, carry):
            for b in range(_NBUF):
                j = g * _NBUF + b

                @pl.when(j <= nch - 1)
                def _():
                    pltpu.make_async_copy(table_sh.at[src_v.at[j]],
                                          rows_v.at[b], gsem[b]).wait()
                    pltpu.async_copy(rows_v.at[b], acc_sh.at[dst_v.at[j]],
                                     ssem[b], add=True)
                    if with_deg:
                        pltpu.async_copy(ones_v, deg_sh.at[dst_v.at[j]],
                                         dsem, add=True)

                        @pl.when(j >= 1)
                        def _():
                            pltpu.make_async_copy(
                                ones_v, deg_sh.at[dst_v.at[0]], dsem).wait()
                    jj = j + _PREF
                    bb = (b + _PREF) % _NBUF

                    @pl.when(jj <= nch - 1)
                    def _():
                        @pl.when(jj >= _NBUF)
                        def _():
                            pltpu.make_async_copy(
                                rows_v.at[bb], acc_sh.at[dst_v.at[0]],
                                ssem[bb]).wait()
                        gather(jj, bb)
            return carry

        lax.fori_loop(0, _NCH0 // _NBUF, step, 0)

        for b in range(_NBUF):
            pltpu.make_async_copy(rows_v.at[b], acc_sh.at[dst_v.at[0]],
                                  ssem[b]).wait()
        if with_deg:
            pltpu.make_async_copy(ones_v, deg_sh.at[dst_v.at[0]], dsem).wait()
        plsc.subcore_barrier()

        pltpu.sync_copy(acc_sh.at[pl.ds(r0, _RPT)],
                        out.at[pl.ds(cid * _NPADR + r0, _RPT)])
        if with_deg:
            @pl.when(sid == 0)
            def _():
                pltpu.sync_copy(deg_sh, deg_out.at[pl.ds(cid * _NPADR,
                                                         _NPADR)])

    return pl.kernel(
        body,
        out_type=tuple(out_type) if with_deg else out_type[0],
        mesh=mesh,
        scratch_types=scratch,
        compiler_params=pltpu.CompilerParams(use_tc_tiling_on_sc=False),
    )


@functools.lru_cache(maxsize=None)
def _get_seg(with_deg):
    return _make_seg(with_deg)


_PBLK = _RPT
_NBLK = _NPADR // _PBLK


def _proj_body(x_ref, w_ref, xl_ref, xr_ref):
    res = jnp.dot(x_ref[...], w_ref[...], preferred_element_type=jnp.float32)
    xl_ref[...] = res[:, :_H]
    xr_ref[...] = res[:, _H:]


def _proj(x, wcat):
    row = lambda i: (i, 0)
    return pl.pallas_call(
        _proj_body,
        grid=(_NBLK,),
        in_specs=[
            pl.BlockSpec((_PBLK, _D_IN), row),
            pl.BlockSpec((_D_IN, 2 * _H), lambda i: (0, 0)),
        ],
        out_specs=[
            pl.BlockSpec((_PBLK, _H), row),
            pl.BlockSpec((_PBLK, _H), row),
        ],
        out_shape=[
            jax.ShapeDtypeStruct((_NPADR, _H), jnp.float32),
            jax.ShapeDtypeStruct((_NPADR, _H), jnp.float32),
        ],
    )(x, wcat)


def _combine_body(p_ref0, p_ref1, d_ref0, d_ref1, xr_ref, b1_ref, h_ref):
    deg = jnp.maximum(d_ref0[...] + d_ref1[...], 1.0)
    mean = (p_ref0[...] + p_ref1[...]) / deg
    h_ref[...] = jnp.maximum(mean + b1_ref[...] + xr_ref[...], 0.0)


def _combine(p, degs2, xr, b1):
    row = lambda i: (i, 0)
    row1 = lambda i: (_NBLK + i, 0)
    return pl.pallas_call(
        _combine_body,
        grid=(_NBLK,),
        in_specs=[
            pl.BlockSpec((_PBLK, _H), row),
            pl.BlockSpec((_PBLK, _H), row1),
            pl.BlockSpec((_PBLK, 1), row),
            pl.BlockSpec((_PBLK, 1), row1),
            pl.BlockSpec((_PBLK, _H), row),
            pl.BlockSpec((1, _H), lambda i: (0, 0)),
        ],
        out_specs=pl.BlockSpec((_PBLK, _H), row),
        out_shape=jax.ShapeDtypeStruct((_NPADR, _H), jnp.float32),
    )(p, p, degs2, degs2, xr, b1)


def _final_body(q_ref0, q_ref1, d_ref0, d_ref1, h_ref, wl_ref, wr_ref,
                b2_ref, o_ref):
    deg = jnp.maximum(d_ref0[...] + d_ref1[...], 1.0)
    mean2 = (q_ref0[...] + q_ref1[...]) / deg
    logits = (jnp.dot(mean2, wl_ref[...], preferred_element_type=jnp.float32)
              + jnp.dot(h_ref[...], wr_ref[...],
                        preferred_element_type=jnp.float32)
              + b2_ref[...])
    m = jnp.max(logits, axis=1, keepdims=True)
    ex = jnp.exp(logits - m)
    s = jnp.sum(ex, axis=1, keepdims=True)
    o_ref[...] = logits - m - jnp.log(s)


def _final(q, degs2, h, wl, wr, b2):
    row = lambda i: (i, 0)
    row1 = lambda i: (_NBLK + i, 0)
    full = lambda i: (0, 0)
    return pl.pallas_call(
        _final_body,
        grid=(_NBLK,),
        in_specs=[
            pl.BlockSpec((_PBLK, _H), row),
            pl.BlockSpec((_PBLK, _H), row1),
            pl.BlockSpec((_PBLK, 1), row),
            pl.BlockSpec((_PBLK, 1), row1),
            pl.BlockSpec((_PBLK, _H), row),
            pl.BlockSpec((_H, _CLS), full),
            pl.BlockSpec((_H, _CLS), full),
            pl.BlockSpec((1, _CLS), full),
        ],
        out_specs=pl.BlockSpec((_PBLK, _CLS), row),
        out_shape=jax.ShapeDtypeStruct((_N, _CLS), jnp.float32),
    )(q, q, degs2, degs2, h, wl, wr, b2)


def kernel(x, edge_index, W1_l, b1, W1_r, W2_l, b2, W2_r):
    src = edge_index[0]
    dst = edge_index[1]
    pad = _EPAD - _E
    srcp = jnp.concatenate([src, jnp.zeros((pad,), jnp.int32)]
                           ).reshape(_NCHT, _CH)
    dstp = jnp.concatenate([dst, jnp.full((pad,), _N, jnp.int32)]
                           ).reshape(_NCHT, _CH)
    z2 = jnp.zeros((_NPADR, _H), jnp.float32)
    z1 = jnp.zeros((_NPADR,), jnp.float32)

    wcat1 = jnp.concatenate([W1_l.T, W1_r.T], axis=1)
    xl, xr = _proj(x, wcat1)

    p, degs = _get_seg(True)(xl, srcp, dstp, z2, z1)
    degs2 = degs.reshape(_NC * _NPADR, 1)
    h = _combine(p, degs2, xr, b1.reshape(1, _H))

    q = _get_seg(False)(h, srcp, dstp, z2)
    return _final(q, degs2, h, W2_l.T, W2_r.T, b2.reshape(1, _CLS))


# speedup vs baseline: 1.4687x; 1.2192x over previous
import functools

import jax
import jax.numpy as jnp
from jax import lax
from jax.experimental import pallas as pl
from jax.experimental.pallas import tpu as pltpu
from jax.experimental.pallas import tpu_sc as plsc

_N = 10000
_E = 160000
_D_IN = 256
_H = 16
_CLS = 41

_NC = 2
_NS = 16
_NW = _NC * _NS
_CH = 128
_EPAD = 163840
_NCHT = _EPAD // _CH
_NCH0 = 40
_NCH1 = 40
_C1OFF = _NS * _NCH0
_NPADR = 10112
_RPT = _NPADR // _NS
_NBUF = 8
_PREF = 3


def _make_seg(with_deg):
    mesh = plsc.VectorSubcoreMesh(core_axis_name="c", subcore_axis_name="s",
                                  num_cores=_NC, num_subcores=_NS)
    out_type = [jax.ShapeDtypeStruct((_NC * _NPADR, _H), jnp.float32)]
    scratch = [
        pltpu.VMEM((_NCH0, _CH), jnp.int32),
        pltpu.VMEM((_NCH0, _CH), jnp.int32),
        pltpu.VMEM((_NBUF, _CH, _H), jnp.float32),
        pltpu.VMEM_SHARED((_NPADR, _H), jnp.float32),
        pltpu.VMEM_SHARED((_NPADR, _H), jnp.float32),
        pltpu.SemaphoreType.DMA,
        [pltpu.SemaphoreType.DMA] * _NBUF,
        [pltpu.SemaphoreType.DMA] * _NBUF,
    ]
    if with_deg:
        out_type.append(jax.ShapeDtypeStruct((_NC * _NPADR, _H), jnp.float32))
        scratch += [
            pltpu.VMEM((_CH, _H), jnp.float32),
            pltpu.VMEM_SHARED((_NPADR, _H), jnp.float32),
            pltpu.SemaphoreType.DMA,
        ]

    def body(*refs):
        if with_deg:
            (table, srcp, dstp, z2, ones_h, out, deg_out,
             src_v, dst_v, rows_v, acc_sh, table_sh, isem, gsem, ssem,
             ones_v, deg_sh, dsem) = refs
        else:
            (table, srcp, dstp, z2, out,
             src_v, dst_v, rows_v, acc_sh, table_sh, isem, gsem, ssem) = refs
        cid = lax.axis_index("c")
        sid = lax.axis_index("s")
        r0 = sid * _RPT
        nch = jnp.where(cid == 0, _NCH0, _NCH1)

        @pl.when(cid == 0)
        def _():
            pltpu.async_copy(srcp.at[pl.ds(sid * _NCH0, _NCH0)],
                             src_v, isem)
            pltpu.async_copy(dstp.at[pl.ds(sid * _NCH0, _NCH0)],
                             dst_v, isem)
            pltpu.make_async_copy(srcp.at[pl.ds(0, _NCH0)], src_v,
                                  isem).wait()
            pltpu.make_async_copy(dstp.at[pl.ds(0, _NCH0)], dst_v,
                                  isem).wait()

        @pl.when(cid == 1)
        def _():
            pltpu.async_copy(srcp.at[pl.ds(_C1OFF + sid * _NCH1, _NCH1)],
                             src_v.at[pl.ds(0, _NCH1)], isem)
            pltpu.async_copy(dstp.at[pl.ds(_C1OFF + sid * _NCH1, _NCH1)],
                             dst_v.at[pl.ds(0, _NCH1)], isem)
            pltpu.make_async_copy(srcp.at[pl.ds(0, _NCH1)],
                                  src_v.at[pl.ds(0, _NCH1)], isem).wait()
            pltpu.make_async_copy(dstp.at[pl.ds(0, _NCH1)],
                                  dst_v.at[pl.ds(0, _NCH1)], isem).wait()
        if with_deg:
            pltpu.async_copy(ones_h, ones_v, isem).wait()
            pltpu.sync_copy(z2.at[pl.ds(r0, _RPT)],
                            deg_sh.at[pl.ds(r0, _RPT)])
        ld_tb = pltpu.async_copy(table.at[pl.ds(r0, _RPT)],
                                 table_sh.at[pl.ds(r0, _RPT)], isem)
        pltpu.sync_copy(z2.at[pl.ds(r0, _RPT)], acc_sh.at[pl.ds(r0, _RPT)])
        ld_tb.wait()
        plsc.subcore_barrier()

        def gather(jj, bb):
            pltpu.async_copy(table_sh.at[src_v.at[jj]], rows_v.at[bb],
                             gsem[bb])

        for b in range(_PREF):
            gather(b, b)

        def step(g, carry):
            for b in range(_NBUF):
                j = g * _NBUF + b

                @pl.when(j <= nch - 1)
                def _():
                    pltpu.make_async_copy(table_sh.at[src_v.at[j]],
                                          rows_v.at[b], gsem[b]).wait()
                    pltpu.async_copy(rows_v.at[b], acc_sh.at[dst_v.at[j]],
                                     ssem[b], add=True)
                    if with_deg:
                        pltpu.async_copy(ones_v, deg_sh.at[dst_v.at[j]],
                                         dsem, add=True)

                        @pl.when(j >= 1)
                        def _():
                            pltpu.make_async_copy(
                                ones_v, deg_sh.at[dst_v.at[0]], dsem).wait()
                    jj = j + _PREF
                    bb = (b + _PREF) % _NBUF

                    @pl.when(jj <= nch - 1)
                    def _():
                        @pl.when(jj >= _NBUF)
                        def _():
                            pltpu.make_async_copy(
                                rows_v.at[bb], acc_sh.at[dst_v.at[0]],
                                ssem[bb]).wait()
                        gather(jj, bb)
            return carry

        lax.fori_loop(0, _NCH0 // _NBUF, step, 0)

        for b in range(_NBUF):
            pltpu.make_async_copy(rows_v.at[b], acc_sh.at[dst_v.at[0]],
                                  ssem[b]).wait()
        if with_deg:
            pltpu.make_async_copy(ones_v, deg_sh.at[dst_v.at[0]], dsem).wait()
        plsc.subcore_barrier()

        pltpu.sync_copy(acc_sh.at[pl.ds(r0, _RPT)],
                        out.at[pl.ds(cid * _NPADR + r0, _RPT)])
        if with_deg:
            pltpu.sync_copy(deg_sh.at[pl.ds(r0, _RPT)],
                            deg_out.at[pl.ds(cid * _NPADR + r0, _RPT)])

    return pl.kernel(
        body,
        out_type=tuple(out_type) if with_deg else out_type[0],
        mesh=mesh,
        scratch_types=scratch,
        compiler_params=pltpu.CompilerParams(use_tc_tiling_on_sc=False),
    )


@functools.lru_cache(maxsize=None)
def _get_seg(with_deg):
    return _make_seg(with_deg)


_PBLK = _RPT
_NBLK = _NPADR // _PBLK


def _proj_body(x_ref, w_ref, xl_ref, xr_ref):
    res = jnp.dot(x_ref[...], w_ref[...], preferred_element_type=jnp.float32)
    xl_ref[...] = res[:, :_H]
    xr_ref[...] = res[:, _H:]


def _proj(x, wcat):
    row = lambda i: (i, 0)
    return pl.pallas_call(
        _proj_body,
        grid=(_NBLK,),
        in_specs=[
            pl.BlockSpec((_PBLK, _D_IN), row),
            pl.BlockSpec((_D_IN, 2 * _H), lambda i: (0, 0)),
        ],
        out_specs=[
            pl.BlockSpec((_PBLK, _H), row),
            pl.BlockSpec((_PBLK, _H), row),
        ],
        out_shape=[
            jax.ShapeDtypeStruct((_NPADR, _H), jnp.float32),
            jax.ShapeDtypeStruct((_NPADR, _H), jnp.float32),
        ],
    )(x, wcat)


_NPK = _NPADR * _H // 128


def _combine_body(p_ref0, p_ref1, d_ref0, d_ref1, xr_ref, b1_ref, h_ref):
    deg = jnp.maximum(d_ref0[...] + d_ref1[...], 1.0)
    mean = (p_ref0[...] + p_ref1[...]) / deg
    h_ref[...] = jnp.maximum(mean + b1_ref[...] + xr_ref[...], 0.0)


def _combine(pp, ddp, xrp, b1p):
    row = lambda i: (i, 0)
    row1 = lambda i: (1 + i, 0)
    return pl.pallas_call(
        _combine_body,
        grid=(1,),
        in_specs=[
            pl.BlockSpec((_NPK, 128), row),
            pl.BlockSpec((_NPK, 128), row1),
            pl.BlockSpec((_NPK, 128), row),
            pl.BlockSpec((_NPK, 128), row1),
            pl.BlockSpec((_NPK, 128), row),
            pl.BlockSpec((1, 128), lambda i: (0, 0)),
        ],
        out_specs=pl.BlockSpec((_NPK, 128), row),
        out_shape=jax.ShapeDtypeStruct((_NPK, 128), jnp.float32),
    )(pp, pp, ddp, ddp, xrp, b1p)


def _final_body(q_ref0, q_ref1, d_ref0, d_ref1, h_ref, wl_ref, wr_ref,
                b2_ref, o_ref):
    deg = jnp.maximum(d_ref0[...] + d_ref1[...], 1.0)
    mean2 = (q_ref0[...] + q_ref1[...]) / deg
    logits = (jnp.dot(mean2, wl_ref[...], preferred_element_type=jnp.float32)
              + jnp.dot(h_ref[...], wr_ref[...],
                        preferred_element_type=jnp.float32)
              + b2_ref[...])
    m = jnp.max(logits, axis=1, keepdims=True)
    ex = jnp.exp(logits - m)
    s = jnp.sum(ex, axis=1, keepdims=True)
    o_ref[...] = logits - m - jnp.log(s)


def _final(q, degs2, h, wl, wr, b2):
    row = lambda i: (i, 0)
    row1 = lambda i: (_NBLK + i, 0)
    full = lambda i: (0, 0)
    return pl.pallas_call(
        _final_body,
        grid=(_NBLK,),
        in_specs=[
            pl.BlockSpec((_PBLK, _H), row),
            pl.BlockSpec((_PBLK, _H), row1),
            pl.BlockSpec((_PBLK, _H), row),
            pl.BlockSpec((_PBLK, _H), row1),
            pl.BlockSpec((_PBLK, _H), row),
            pl.BlockSpec((_H, _CLS), full),
            pl.BlockSpec((_H, _CLS), full),
            pl.BlockSpec((1, _CLS), full),
        ],
        out_specs=pl.BlockSpec((_PBLK, _CLS), row),
        out_shape=jax.ShapeDtypeStruct((_N, _CLS), jnp.float32),
    )(q, q, degs2, degs2, h, wl, wr, b2)


def kernel(x, edge_index, W1_l, b1, W1_r, W2_l, b2, W2_r):
    src = edge_index[0]
    dst = edge_index[1]
    pad = _EPAD - _E
    srcp = jnp.concatenate([src, jnp.zeros((pad,), jnp.int32)]
                           ).reshape(_NCHT, _CH)
    dstp = jnp.concatenate([dst, jnp.full((pad,), _N, jnp.int32)]
                           ).reshape(_NCHT, _CH)
    z2 = jnp.zeros((_NPADR, _H), jnp.float32)
    ones_h = jnp.ones((_CH, _H), jnp.float32)

    wcat1 = jnp.concatenate([W1_l.T, W1_r.T], axis=1)
    xl, xr = _proj(x, wcat1)

    p, degs = _get_seg(True)(xl, srcp, dstp, z2, ones_h)
    pp = p.reshape(_NC * _NPK, 128)
    ddp = degs.reshape(_NC * _NPK, 128)
    xrp = xr.reshape(_NPK, 128)
    b1p = jnp.tile(b1, 8).reshape(1, 128)
    hp = _combine(pp, ddp, xrp, b1p)
    h = hp.reshape(_NPADR, _H)

    q = _get_seg(False)(h, srcp, dstp, z2)
    return _final(q, degs, h, W2_l.T, W2_r.T, b2.reshape(1, _CLS))


# speedup vs baseline: 1.4890x; 1.0139x over previous
import functools

import jax
import jax.numpy as jnp
from jax import lax
from jax.experimental import pallas as pl
from jax.experimental.pallas import tpu as pltpu
from jax.experimental.pallas import tpu_sc as plsc

_N = 10000
_E = 160000
_D_IN = 256
_H = 16
_CLS = 41

_NC = 2
_NS = 16
_NW = _NC * _NS
_CH = 128
_EPAD = 163840
_NCHT = _EPAD // _CH
_NCH0 = 44
_NCH1 = 36
_NCHB = 48
_C1OFF = _NS * _NCH0
_NPADR = 10112
_RPT = _NPADR // _NS
_NBUF = 8
_PREF = 3


def _make_seg(with_deg):
    mesh = plsc.VectorSubcoreMesh(core_axis_name="c", subcore_axis_name="s",
                                  num_cores=_NC, num_subcores=_NS)
    out_type = [jax.ShapeDtypeStruct((_NC * _NPADR, _H), jnp.float32)]
    scratch = [
        pltpu.VMEM((_NCHB, _CH), jnp.int32),
        pltpu.VMEM((_NCHB, _CH), jnp.int32),
        pltpu.VMEM((_NBUF, _CH, _H), jnp.float32),
        pltpu.VMEM_SHARED((_NPADR, _H), jnp.float32),
        pltpu.VMEM_SHARED((_NPADR, _H), jnp.float32),
        pltpu.SemaphoreType.DMA,
        [pltpu.SemaphoreType.DMA] * _NBUF,
        [pltpu.SemaphoreType.DMA] * _NBUF,
    ]
    if with_deg:
        out_type.append(jax.ShapeDtypeStruct((_NC * _NPADR, _H), jnp.float32))
        scratch += [
            pltpu.VMEM((_CH, _H), jnp.float32),
            pltpu.VMEM_SHARED((_NPADR, _H), jnp.float32),
            pltpu.SemaphoreType.DMA,
        ]

    def body(*refs):
        if with_deg:
            (table, srcp, dstp, z2, ones_h, out, deg_out,
             src_v, dst_v, rows_v, acc_sh, table_sh, isem, gsem, ssem,
             ones_v, deg_sh, dsem) = refs
        else:
            (table, srcp, dstp, z2, out,
             src_v, dst_v, rows_v, acc_sh, table_sh, isem, gsem, ssem) = refs
        cid = lax.axis_index("c")
        sid = lax.axis_index("s")
        r0 = sid * _RPT
        nch = jnp.where(cid == 0, _NCH0, _NCH1)

        @pl.when(cid == 0)
        def _():
            pltpu.async_copy(srcp.at[pl.ds(sid * _NCH0, _NCH0)],
                             src_v.at[pl.ds(0, _NCH0)], isem)
            pltpu.async_copy(dstp.at[pl.ds(sid * _NCH0, _NCH0)],
                             dst_v.at[pl.ds(0, _NCH0)], isem)
            pltpu.make_async_copy(srcp.at[pl.ds(0, _NCH0)],
                                  src_v.at[pl.ds(0, _NCH0)], isem).wait()
            pltpu.make_async_copy(dstp.at[pl.ds(0, _NCH0)],
                                  dst_v.at[pl.ds(0, _NCH0)], isem).wait()

        @pl.when(cid == 1)
        def _():
            pltpu.async_copy(srcp.at[pl.ds(_C1OFF + sid * _NCH1, _NCH1)],
                             src_v.at[pl.ds(0, _NCH1)], isem)
            pltpu.async_copy(dstp.at[pl.ds(_C1OFF + sid * _NCH1, _NCH1)],
                             dst_v.at[pl.ds(0, _NCH1)], isem)
            pltpu.make_async_copy(srcp.at[pl.ds(0, _NCH1)],
                                  src_v.at[pl.ds(0, _NCH1)], isem).wait()
            pltpu.make_async_copy(dstp.at[pl.ds(0, _NCH1)],
                                  dst_v.at[pl.ds(0, _NCH1)], isem).wait()
        if with_deg:
            pltpu.async_copy(ones_h, ones_v, isem).wait()
            pltpu.sync_copy(z2.at[pl.ds(r0, _RPT)],
                            deg_sh.at[pl.ds(r0, _RPT)])
        ld_tb = pltpu.async_copy(table.at[pl.ds(r0, _RPT)],
                                 table_sh.at[pl.ds(r0, _RPT)], isem)
        pltpu.sync_copy(z2.at[pl.ds(r0, _RPT)], acc_sh.at[pl.ds(r0, _RPT)])
        ld_tb.wait()
        plsc.subcore_barrier()

        def gather(jj, bb):
            pltpu.async_copy(table_sh.at[src_v.at[jj]], rows_v.at[bb],
                             gsem[bb])

        for b in range(_PREF):
            gather(b, b)

        def step(g, carry):
            for b in range(_NBUF):
                j = g * _NBUF + b

                @pl.when(j <= nch - 1)
                def _():
                    pltpu.make_async_copy(table_sh.at[src_v.at[j]],
                                          rows_v.at[b], gsem[b]).wait()
                    pltpu.async_copy(rows_v.at[b], acc_sh.at[dst_v.at[j]],
                                     ssem[b], add=True)
                    if with_deg:
                        pltpu.async_copy(ones_v, deg_sh.at[dst_v.at[j]],
                                         dsem, add=True)

                        @pl.when(j >= 1)
                        def _():
                            pltpu.make_async_copy(
                                ones_v, deg_sh.at[dst_v.at[0]], dsem).wait()
                    jj = j + _PREF
                    bb = (b + _PREF) % _NBUF

                    @pl.when(jj <= nch - 1)
                    def _():
                        @pl.when(jj >= _NBUF)
                        def _():
                            pltpu.make_async_copy(
                                rows_v.at[bb], acc_sh.at[dst_v.at[0]],
                                ssem[bb]).wait()
                        gather(jj, bb)
            return carry

        lax.fori_loop(0, _NCHB // _NBUF, step, 0)

        for b in range(_NBUF):
            pltpu.make_async_copy(rows_v.at[b], acc_sh.at[dst_v.at[0]],
                                  ssem[b]).wait()
        if with_deg:
            pltpu.make_async_copy(ones_v, deg_sh.at[dst_v.at[0]], dsem).wait()
        plsc.subcore_barrier()

        pltpu.sync_copy(acc_sh.at[pl.ds(r0, _RPT)],
                        out.at[pl.ds(cid * _NPADR + r0, _RPT)])
        if with_deg:
            pltpu.sync_copy(deg_sh.at[pl.ds(r0, _RPT)],
                            deg_out.at[pl.ds(cid * _NPADR + r0, _RPT)])

    return pl.kernel(
        body,
        out_type=tuple(out_type) if with_deg else out_type[0],
        mesh=mesh,
        scratch_types=scratch,
        compiler_params=pltpu.CompilerParams(use_tc_tiling_on_sc=False),
    )


@functools.lru_cache(maxsize=None)
def _get_seg(with_deg):
    return _make_seg(with_deg)


_PBLK = _RPT
_NBLK = _NPADR // _PBLK


_NPK = _NPADR * _H // 128
_PJB = _NPADR // 2
_PKB = _NPK // 2


def _proj_body(x_ref, w_ref, xl_ref, xr_ref):
    res = jnp.dot(x_ref[...], w_ref[...], preferred_element_type=jnp.float32)
    xl_ref[...] = res[:, :_H]
    xr_ref[...] = res[:, _H:]


def _proj(x, wcat):
    row = lambda i: (i, 0)
    return pl.pallas_call(
        _proj_body,
        grid=(_NBLK,),
        in_specs=[
            pl.BlockSpec((_PBLK, _D_IN), row),
            pl.BlockSpec((_D_IN, 2 * _H), lambda i: (0, 0)),
        ],
        out_specs=[
            pl.BlockSpec((_PBLK, _H), row),
            pl.BlockSpec((_PBLK, _H), row),
        ],
        out_shape=[
            jax.ShapeDtypeStruct((_NPADR, _H), jnp.float32),
            jax.ShapeDtypeStruct((_NPADR, _H), jnp.float32),
        ],
    )(x, wcat)


def _combine_body(p_ref0, p_ref1, d_ref0, d_ref1, xr_ref, b1_ref, h_ref):
    deg = jnp.maximum(d_ref0[...] + d_ref1[...], 1.0)
    mean = (p_ref0[...] + p_ref1[...]) / deg
    h_ref[...] = jnp.maximum(mean + b1_ref[...] + xr_ref[...], 0.0)


def _combine(pp, ddp, xrp, b1p):
    row = lambda i: (i, 0)
    row1 = lambda i: (1 + i, 0)
    return pl.pallas_call(
        _combine_body,
        grid=(1,),
        in_specs=[
            pl.BlockSpec((_NPK, 128), row),
            pl.BlockSpec((_NPK, 128), row1),
            pl.BlockSpec((_NPK, 128), row),
            pl.BlockSpec((_NPK, 128), row1),
            pl.BlockSpec((_NPK, 128), row),
            pl.BlockSpec((1, 128), lambda i: (0, 0)),
        ],
        out_specs=pl.BlockSpec((_NPK, 128), row),
        out_shape=jax.ShapeDtypeStruct((_NPK, 128), jnp.float32),
    )(pp, pp, ddp, ddp, xrp, b1p)


def _final_body(q_ref0, q_ref1, d_ref0, d_ref1, h_ref, wl_ref, wr_ref,
                b2_ref, o_ref):
    deg = jnp.maximum(d_ref0[...] + d_ref1[...], 1.0)
    mean2 = (q_ref0[...] + q_ref1[...]) / deg
    logits = (jnp.dot(mean2, wl_ref[...], preferred_element_type=jnp.float32)
              + jnp.dot(h_ref[...], wr_ref[...],
                        preferred_element_type=jnp.float32)
              + b2_ref[...])
    m = jnp.max(logits, axis=1, keepdims=True)
    ex = jnp.exp(logits - m)
    s = jnp.sum(ex, axis=1, keepdims=True)
    o_ref[...] = logits - m - jnp.log(s)


def _final(q, degs, h, wl, wr, b2):
    row = lambda i: (i, 0)
    row1 = lambda i: (_NBLK + i, 0)
    full = lambda i: (0, 0)
    return pl.pallas_call(
        _final_body,
        grid=(_NBLK,),
        in_specs=[
            pl.BlockSpec((_PBLK, _H), row),
            pl.BlockSpec((_PBLK, _H), row1),
            pl.BlockSpec((_PBLK, _H), row),
            pl.BlockSpec((_PBLK, _H), row1),
            pl.BlockSpec((_PBLK, _H), row),
            pl.BlockSpec((_H, _CLS), full),
            pl.BlockSpec((_H, _CLS), full),
            pl.BlockSpec((1, _CLS), full),
        ],
        out_specs=pl.BlockSpec((_PBLK, _CLS), row),
        out_shape=jax.ShapeDtypeStruct((_N, _CLS), jnp.float32),
    )(q, q, degs, degs, h, wl, wr, b2)


def kernel(x, edge_index, W1_l, b1, W1_r, W2_l, b2, W2_r):
    src = edge_index[0]
    dst = edge_index[1]
    pad = _EPAD - _E
    srcp = jnp.concatenate([src, jnp.zeros((pad,), jnp.int32)]
                           ).reshape(_NCHT, _CH)
    dstp = jnp.concatenate([dst, jnp.full((pad,), _N, jnp.int32)]
                           ).reshape(_NCHT, _CH)
    z2 = jnp.zeros((_NPADR, _H), jnp.float32)
    ones_h = jnp.ones((_CH, _H), jnp.float32)

    wcat1 = jnp.concatenate([W1_l.T, W1_r.T], axis=1)
    xl, xr = _proj(x, wcat1)
    xrp = xr.reshape(_NPK, 128)

    p, degs = _get_seg(True)(xl, srcp, dstp, z2, ones_h)
    pp = p.reshape(_NC * _NPK, 128)
    ddp = degs.reshape(_NC * _NPK, 128)
    b1p = jnp.tile(b1, 8).reshape(1, 128)
    hp = _combine(pp, ddp, xrp, b1p)
    h = hp.reshape(_NPADR, _H)

    q = _get_seg(False)(h, srcp, dstp, z2)
    return _final(q, degs, h, W2_l.T, W2_r.T, b2.reshape(1, _CLS))


# speedup vs baseline: 1.4900x; 1.0006x over previous
import functools

import jax
import jax.numpy as jnp
from jax import lax
from jax.experimental import pallas as pl
from jax.experimental.pallas import tpu as pltpu
from jax.experimental.pallas import tpu_sc as plsc

_N = 10000
_E = 160000
_D_IN = 256
_H = 16
_CLS = 41

_NC = 2
_NS = 16
_NW = _NC * _NS
_CH = 128
_EPAD = 163840
_NCHT = _EPAD // _CH
_NCH0 = 44
_NCH1 = 36
_NCHB = 48
_C1OFF = _NS * _NCH0
_NPADR = 10112
_RPT = _NPADR // _NS
_NBUF = 8
_PREF = 3


def _make_seg(with_deg):
    mesh = plsc.VectorSubcoreMesh(core_axis_name="c", subcore_axis_name="s",
                                  num_cores=_NC, num_subcores=_NS)
    out_type = [jax.ShapeDtypeStruct((_NC * _NPADR, _H), jnp.float32)]
    scratch = [
        pltpu.VMEM((_NCHB, _CH), jnp.int32),
        pltpu.VMEM((_NCHB, _CH), jnp.int32),
        pltpu.VMEM((_NBUF, _CH, _H), jnp.float32),
        pltpu.VMEM_SHARED((_NPADR, _H), jnp.float32),
        pltpu.VMEM_SHARED((_NPADR, _H), jnp.float32),
        pltpu.SemaphoreType.DMA,
        [pltpu.SemaphoreType.DMA] * _NBUF,
        [pltpu.SemaphoreType.DMA] * _NBUF,
    ]
    if with_deg:
        out_type.append(jax.ShapeDtypeStruct((_NC * _NPADR, _H), jnp.float32))
        scratch += [
            pltpu.VMEM((_CH, _H), jnp.float32),
            pltpu.VMEM_SHARED((_NPADR, _H), jnp.float32),
            pltpu.SemaphoreType.DMA,
        ]

    def body(*refs):
        if with_deg:
            (table, srcp, dstp, z2, ones_h, out, deg_out,
             src_v, dst_v, rows_v, acc_sh, table_sh, isem, gsem, ssem,
             ones_v, deg_sh, dsem) = refs
        else:
            (table, srcp, dstp, z2, out,
             src_v, dst_v, rows_v, acc_sh, table_sh, isem, gsem, ssem) = refs
        cid = lax.axis_index("c")
        sid = lax.axis_index("s")
        r0 = sid * _RPT
        nch = jnp.where(cid == 0, _NCH0, _NCH1)

        @pl.when(cid == 0)
        def _():
            pltpu.async_copy(srcp.at[pl.ds(sid * _NCH0, _NCH0)],
                             src_v.at[pl.ds(0, _NCH0)], isem)
            pltpu.async_copy(dstp.at[pl.ds(sid * _NCH0, _NCH0)],
                             dst_v.at[pl.ds(0, _NCH0)], isem)
            pltpu.make_async_copy(srcp.at[pl.ds(0, _NCH0)],
                                  src_v.at[pl.ds(0, _NCH0)], isem).wait()
            pltpu.make_async_copy(dstp.at[pl.ds(0, _NCH0)],
                                  dst_v.at[pl.ds(0, _NCH0)], isem).wait()

        @pl.when(cid == 1)
        def _():
            pltpu.async_copy(srcp.at[pl.ds(_C1OFF + sid * _NCH1, _NCH1)],
                             src_v.at[pl.ds(0, _NCH1)], isem)
            pltpu.async_copy(dstp.at[pl.ds(_C1OFF + sid * _NCH1, _NCH1)],
                             dst_v.at[pl.ds(0, _NCH1)], isem)
            pltpu.make_async_copy(srcp.at[pl.ds(0, _NCH1)],
                                  src_v.at[pl.ds(0, _NCH1)], isem).wait()
            pltpu.make_async_copy(dstp.at[pl.ds(0, _NCH1)],
                                  dst_v.at[pl.ds(0, _NCH1)], isem).wait()
        if with_deg:
            pltpu.async_copy(ones_h, ones_v, isem).wait()
            pltpu.sync_copy(z2.at[pl.ds(r0, _RPT)],
                            deg_sh.at[pl.ds(r0, _RPT)])
        ld_tb = pltpu.async_copy(table.at[pl.ds(r0, _RPT)],
                                 table_sh.at[pl.ds(r0, _RPT)], isem)
        pltpu.sync_copy(z2.at[pl.ds(r0, _RPT)], acc_sh.at[pl.ds(r0, _RPT)])
        ld_tb.wait()
        plsc.subcore_barrier()

        def gather(jj, bb):
            pltpu.async_copy(table_sh.at[src_v.at[jj]], rows_v.at[bb],
                             gsem[bb])

        for b in range(_PREF):
            gather(b, b)

        def step(g, carry):
            for b in range(_NBUF):
                j = g * _NBUF + b

                @pl.when(j <= nch - 1)
                def _():
                    pltpu.make_async_copy(table_sh.at[src_v.at[j]],
                                          rows_v.at[b], gsem[b]).wait()
                    pltpu.async_copy(rows_v.at[b], acc_sh.at[dst_v.at[j]],
                                     ssem[b], add=True)
                    if with_deg:
                        pltpu.async_copy(ones_v, deg_sh.at[dst_v.at[j]],
                                         dsem, add=True)

                        @pl.when(j >= 1)
                        def _():
                            pltpu.make_async_copy(
                                ones_v, deg_sh.at[dst_v.at[0]], dsem).wait()
                    jj = j + _PREF
                    bb = (b + _PREF) % _NBUF

                    @pl.when(jj <= nch - 1)
                    def _():
                        @pl.when(jj >= _NBUF)
                        def _():
                            pltpu.make_async_copy(
                                rows_v.at[bb], acc_sh.at[dst_v.at[0]],
                                ssem[bb]).wait()
                        gather(jj, bb)
            return carry

        lax.fori_loop(0, _NCHB // _NBUF, step, 0)

        for b in range(_NBUF):
            pltpu.make_async_copy(rows_v.at[b], acc_sh.at[dst_v.at[0]],
                                  ssem[b]).wait()
        if with_deg:
            pltpu.make_async_copy(ones_v, deg_sh.at[dst_v.at[0]], dsem).wait()
        plsc.subcore_barrier()

        pltpu.sync_copy(acc_sh.at[pl.ds(r0, _RPT)],
                        out.at[pl.ds(cid * _NPADR + r0, _RPT)])
        if with_deg:
            pltpu.sync_copy(deg_sh.at[pl.ds(r0, _RPT)],
                            deg_out.at[pl.ds(cid * _NPADR + r0, _RPT)])

    return pl.kernel(
        body,
        out_type=tuple(out_type) if with_deg else out_type[0],
        mesh=mesh,
        scratch_types=scratch,
        compiler_params=pltpu.CompilerParams(use_tc_tiling_on_sc=False),
    )


@functools.lru_cache(maxsize=None)
def _get_seg(with_deg):
    return _make_seg(with_deg)


_PBLK = _RPT
_NBLK = _NPADR // _PBLK


_NPK = _NPADR * _H // 128


def _proj_body(x_ref, w_ref, xl_ref, xr_ref):
    res = jnp.dot(x_ref[...], w_ref[...], preferred_element_type=jnp.float32)
    xl_ref[...] = res[:, :_H]
    xr_ref[...] = res[:, _H:]


def _proj(x, wcat):
    row = lambda i: (i, 0)
    return pl.pallas_call(
        _proj_body,
        grid=(_NBLK,),
        in_specs=[
            pl.BlockSpec((_PBLK, _D_IN), row),
            pl.BlockSpec((_D_IN, 2 * _H), lambda i: (0, 0)),
        ],
        out_specs=[
            pl.BlockSpec((_PBLK, _H), row),
            pl.BlockSpec((_PBLK, _H), row),
        ],
        out_shape=[
            jax.ShapeDtypeStruct((_NPADR, _H), jnp.float32),
            jax.ShapeDtypeStruct((_NPADR, _H), jnp.float32),
        ],
    )(x, wcat)


def _combine_body(p_ref0, p_ref1, d_ref0, d_ref1, xr_ref, b1_ref, h_ref):
    deg = jnp.maximum(d_ref0[...] + d_ref1[...], 1.0)
    mean = (p_ref0[...] + p_ref1[...]) / deg
    h_ref[...] = jnp.maximum(mean + b1_ref[...] + xr_ref[...], 0.0)


def _combine(pp, ddp, xrp, b1p):
    row = lambda i: (i, 0)
    row1 = lambda i: (1 + i, 0)
    return pl.pallas_call(
        _combine_body,
        grid=(1,),
        in_specs=[
            pl.BlockSpec((_NPK, 128), row),
            pl.BlockSpec((_NPK, 128), row1),
            pl.BlockSpec((_NPK, 128), row),
            pl.BlockSpec((_NPK, 128), row1),
            pl.BlockSpec((_NPK, 128), row),
            pl.BlockSpec((1, 128), lambda i: (0, 0)),
        ],
        out_specs=pl.BlockSpec((_NPK, 128), row),
        out_shape=jax.ShapeDtypeStruct((_NPK, 128), jnp.float32),
    )(pp, pp, ddp, ddp, xrp, b1p)


def _final_body(q_ref0, q_ref1, d_ref0, d_ref1, h_ref, wl_ref, wr_ref,
                b2_ref, o_ref):
    deg = jnp.maximum(d_ref0[...] + d_ref1[...], 1.0)
    mean2 = (q_ref0[...] + q_ref1[...]) / deg
    logits = (jnp.dot(mean2, wl_ref[...], preferred_element_type=jnp.float32)
              + jnp.dot(h_ref[...], wr_ref[...],
                        preferred_element_type=jnp.float32)
              + b2_ref[...])
    m = jnp.max(logits, axis=1, keepdims=True)
    ex = jnp.exp(logits - m)
    s = jnp.sum(ex, axis=1, keepdims=True)
    o_ref[...] = logits - m - jnp.log(s)


def _final(q, degs, h, wl, wr, b2):
    row = lambda i: (i, 0)
    row1 = lambda i: (_NBLK + i, 0)
    full = lambda i: (0, 0)
    return pl.pallas_call(
        _final_body,
        grid=(_NBLK,),
        in_specs=[
            pl.BlockSpec((_PBLK, _H), row),
            pl.BlockSpec((_PBLK, _H), row1),
            pl.BlockSpec((_PBLK, _H), row),
            pl.BlockSpec((_PBLK, _H), row1),
            pl.BlockSpec((_PBLK, _H), row),
            pl.BlockSpec((_H, _CLS), full),
            pl.BlockSpec((_H, _CLS), full),
            pl.BlockSpec((1, _CLS), full),
        ],
        out_specs=pl.BlockSpec((_PBLK, _CLS), row),
        out_shape=jax.ShapeDtypeStruct((_N, _CLS), jnp.float32),
    )(q, q, degs, degs, h, wl, wr, b2)


def kernel(x, edge_index, W1_l, b1, W1_r, W2_l, b2, W2_r):
    src = edge_index[0]
    dst = edge_index[1]
    pad = _EPAD - _E
    srcp = jnp.concatenate([src, jnp.zeros((pad,), jnp.int32)]
                           ).reshape(_NCHT, _CH)
    dstp = jnp.concatenate([dst, jnp.full((pad,), _N, jnp.int32)]
                           ).reshape(_NCHT, _CH)
    z2 = jnp.zeros((_NPADR, _H), jnp.float32)
    ones_h = jnp.ones((_CH, _H), jnp.float32)

    wcat1 = jnp.concatenate([W1_l.T, W1_r.T], axis=1)
    xl, xr = _proj(x, wcat1)
    xrp = xr.reshape(_NPK, 128)

    p, degs = _get_seg(True)(xl, srcp, dstp, z2, ones_h)
    pp = p.reshape(_NC * _NPK, 128)
    ddp = degs.reshape(_NC * _NPK, 128)
    b1p = jnp.tile(b1, 8).reshape(1, 128)
    hp = _combine(pp, ddp, xrp, b1p)
    h = hp.reshape(_NPADR, _H)

    q = _get_seg(False)(h, srcp, dstp, z2)
    return _final(q, degs, h, W2_l.T, W2_r.T, b2.reshape(1, _CLS))
